# baseline (speedup 1.0000x reference)
import jax
import jax.numpy as jnp
from jax import lax
from jax.experimental import pallas as pl
from jax.experimental.pallas import tpu as pltpu
from jax.experimental.pallas import tpu_sc as plsc

B = 16384
T = 26
D = 16

NC = 2
NS = 16
NW = NC * NS
BPW = B // NW
C = 64
NCHUNK = BPW // C
NG = C // 16


def _body(user_hbm, item_hbm, uemb_hbm, iemb_hbm, bc_hbm, i4u_hbm, i4i_hbm,
          out_hbm, idx_u, idx_i, i4u, i4i, urows, irows, bcbuf, bc65,
          outv, sem):
  wid = lax.axis_index("s") * NC + lax.axis_index("c")
  base = wid * BPW

  pltpu.sync_copy(user_hbm.at[pl.ds(base, BPW)], idx_u)
  pltpu.sync_copy(item_hbm.at[pl.ds(base, BPW)], idx_i)
  pltpu.sync_copy(i4u_hbm.at[pl.ds(base * 4, BPW * 4)], i4u)
  pltpu.sync_copy(i4i_hbm.at[pl.ds(base * 4, BPW * 4)], i4i)

  iota = lax.iota(jnp.int32, 16)
  rot = [jnp.bitwise_and(iota + d, 15) for d in range(D)]

  @pl.loop(0, NCHUNK)
  def _chunk(c):
    cps = []
    for h in range(2):
      sl = pl.ds(c * 4 * C + h * 128, 128)
      dst = pl.ds(h * 128, 128)
      cps.append(pltpu.async_copy(uemb_hbm.at[i4u.at[sl]],
                                  urows.at[dst, :], sem))
      cps.append(pltpu.async_copy(iemb_hbm.at[i4i.at[sl]],
                                  irows.at[dst, :], sem))
    cps.append(
        pltpu.async_copy(bc_hbm.at[pl.ds(base + c * C, C), :], bcbuf, sem))
    for cp in cps:
      cp.wait()

    @pl.loop(0, C)
    def _repack(l):
      for k in range(4):
        v = plsc.load_gather(bcbuf, [jnp.full((16,), l, jnp.int32),
                                     iota + k * 16])
        plsc.store_scatter(bc65, [l * 65 + k * 16 + iota], v)

    @pl.loop(0, NG)
    def _group(g):
      rvec = iota + g * 16
      bbase = rvec * 65
      uvec = idx_u[pl.ds(c * C + g * 16, 16)]
      ivec = idx_i[pl.ds(c * C + g * 16, 16)]
      au = rvec * 512 + uvec * 416 - jnp.right_shift(uvec * 13, 2) * 128
      ai = rvec * 512 + ivec * 416 - jnp.right_shift(ivec * 13, 2) * 128

      @pl.loop(0, T, init_carry=jnp.zeros((16,), jnp.float32), unroll=2)
      def _tag(t, acc):
        aut = au + t * D
        ait = ai + t * D
        ps = [jnp.zeros((16,), jnp.float32) for _ in range(4)]
        for d in range(D):
          fu = aut + rot[d]
          fi = ait + rot[d]
          uu = plsc.load_gather(urows, [jnp.right_shift(fu, 7),
                                        jnp.bitwise_and(fu, 127)])
          iv = plsc.load_gather(irows, [jnp.right_shift(fi, 7),
                                        jnp.bitwise_and(fi, 127)])
          ps[d % 4] = ps[d % 4] + uu * iv
        s = (ps[0] + ps[1]) + (ps[2] + ps[3])
        s = s + plsc.load_gather(bc65, [bbase + t])
        sig = 1.0 / (1.0 + jnp.exp(-s))
        cf = plsc.load_gather(bc65, [bbase + (32 + t)])
        return acc + sig * cf

      outv[pl.ds(c * C + g * 16, 16)] = _tag

  pltpu.sync_copy(outv, out_hbm.at[pl.ds(base, BPW)])


@jax.jit
def _run(user, item, uemb, iemb, bc, i4u, i4i):
  mesh = plsc.VectorSubcoreMesh(core_axis_name="c", subcore_axis_name="s")
  f = pl.kernel(
      _body,
      out_type=jax.ShapeDtypeStruct((B,), jnp.float32),
      mesh=mesh,
      scratch_types=[
          pltpu.VMEM((BPW,), jnp.int32),
          pltpu.VMEM((BPW,), jnp.int32),
          pltpu.VMEM((4 * BPW,), jnp.int32),
          pltpu.VMEM((4 * BPW,), jnp.int32),
          pltpu.VMEM((4 * C, 128), jnp.float32),
          pltpu.VMEM((4 * C, 128), jnp.float32),
          pltpu.VMEM((C, 128), jnp.float32),
          pltpu.VMEM((C * 65,), jnp.float32),
          pltpu.VMEM((BPW,), jnp.float32),
          pltpu.SemaphoreType.DMA,
      ],
      compiler_params=pltpu.CompilerParams(use_tc_tiling_on_sc=False,
                                           needs_layout_passes=False),
  )
  return f(user, item, uemb, iemb, bc, i4u, i4i)


def kernel(user, item, user_tag_embeddings, item_tag_embeddings,
           user_aspect_bias, item_aspect_bias, global_aspect_bias,
           user_coeff, item_coeff, global_coeff):
  user = user.astype(jnp.int32)
  item = item.astype(jnp.int32)
  bsum = (jnp.take(user_aspect_bias, user, axis=0)
          + jnp.take(item_aspect_bias, item, axis=0)
          + global_aspect_bias)
  csum = (jnp.take(user_coeff, user, axis=0)
          + jnp.take(item_coeff, item, axis=0)
          + global_coeff)
  z6 = jnp.zeros((B, 32 - T), jnp.float32)
  z70 = jnp.zeros((B, 128 - 32 - T), jnp.float32)
  bc = jnp.concatenate([bsum, z6, csum, z70], axis=1)
  uemb = user_tag_embeddings.reshape(-1, 128)
  iemb = item_tag_embeddings.reshape(-1, 128)
  i4u = (jnp.right_shift(user * 13, 2)[:, None]
         + jnp.arange(4, dtype=jnp.int32)[None, :]).reshape(-1)
  i4i = (jnp.right_shift(item * 13, 2)[:, None]
         + jnp.arange(4, dtype=jnp.int32)[None, :]).reshape(-1)
  return _run(user, item, uemb, iemb, bc, i4u, i4i)


# speedup vs baseline: 1.1176x; 1.1176x over previous
import jax
import jax.numpy as jnp
from jax import lax
from jax.experimental import pallas as pl
from jax.experimental.pallas import tpu as pltpu
from jax.experimental.pallas import tpu_sc as plsc

B = 16384
T = 26
D = 16

NC = 2
NS = 16
NW = NC * NS
BPW = B // NW
C = 64
NCHUNK = BPW // C
NG = C // 16


def _body(user_hbm, item_hbm, uemb_hbm, iemb_hbm, bc_hbm,
          out_hbm, idx_u, idx_i, urows, irows, bcbuf, bc65,
          outv, sem):
  wid = lax.axis_index("s") * NC + lax.axis_index("c")
  base = wid * BPW

  pltpu.sync_copy(user_hbm.at[pl.ds(base, BPW)], idx_u)
  pltpu.sync_copy(item_hbm.at[pl.ds(base, BPW)], idx_i)

  iota = lax.iota(jnp.int32, 16)
  rot = [jnp.bitwise_and(iota + d, 15) for d in range(D)]

  @pl.loop(0, NCHUNK)
  def _chunk(c):
    iu = idx_u.at[pl.ds(c * C, C)]
    ii = idx_i.at[pl.ds(c * C, C)]
    cps = [
        pltpu.async_copy(uemb_hbm.at[iu], urows, sem),
        pltpu.async_copy(iemb_hbm.at[ii], irows, sem),
        pltpu.async_copy(bc_hbm.at[pl.ds(base + c * C, C), :], bcbuf, sem),
    ]
    for cp in cps:
      cp.wait()

    @pl.loop(0, C)
    def _repack(l):
      for k in range(4):
        v = plsc.load_gather(bcbuf, [jnp.full((16,), l, jnp.int32),
                                     iota + k * 16])
        plsc.store_scatter(bc65, [l * 65 + k * 16 + iota], v)

    @pl.loop(0, NG)
    def _group(g):
      rvec = iota + g * 16
      bbase = rvec * 65

      @pl.loop(0, T, init_carry=jnp.zeros((16,), jnp.float32), unroll=2)
      def _tag(t, acc):
        ps = [jnp.zeros((16,), jnp.float32) for _ in range(4)]
        for d in range(D):
          cvec = rot[d] + t * D
          uu = plsc.load_gather(urows, [rvec, cvec])
          iv = plsc.load_gather(irows, [rvec, cvec])
          ps[d % 4] = ps[d % 4] + uu * iv
        s = (ps[0] + ps[1]) + (ps[2] + ps[3])
        s = s + plsc.load_gather(bc65, [bbase + t])
        sig = 1.0 / (1.0 + jnp.exp(-s))
        cf = plsc.load_gather(bc65, [bbase + (32 + t)])
        return acc + sig * cf

      outv[pl.ds(c * C + g * 16, 16)] = _tag

  pltpu.sync_copy(outv, out_hbm.at[pl.ds(base, BPW)])


@jax.jit
def _run(user, item, uemb, iemb, bc):
  mesh = plsc.VectorSubcoreMesh(core_axis_name="c", subcore_axis_name="s")
  f = pl.kernel(
      _body,
      out_type=jax.ShapeDtypeStruct((B,), jnp.float32),
      mesh=mesh,
      scratch_types=[
          pltpu.VMEM((BPW,), jnp.int32),
          pltpu.VMEM((BPW,), jnp.int32),
          pltpu.VMEM((C, T * D), jnp.float32),
          pltpu.VMEM((C, T * D), jnp.float32),
          pltpu.VMEM((C, 128), jnp.float32),
          pltpu.VMEM((C * 65,), jnp.float32),
          pltpu.VMEM((BPW,), jnp.float32),
          pltpu.SemaphoreType.DMA,
      ],
      compiler_params=pltpu.CompilerParams(use_tc_tiling_on_sc=False,
                                           needs_layout_passes=False),
  )
  return f(user, item, uemb, iemb, bc)


def kernel(user, item, user_tag_embeddings, item_tag_embeddings,
           user_aspect_bias, item_aspect_bias, global_aspect_bias,
           user_coeff, item_coeff, global_coeff):
  user = user.astype(jnp.int32)
  item = item.astype(jnp.int32)
  bsum = (jnp.take(user_aspect_bias, user, axis=0)
          + jnp.take(item_aspect_bias, item, axis=0)
          + global_aspect_bias)
  csum = (jnp.take(user_coeff, user, axis=0)
          + jnp.take(item_coeff, item, axis=0)
          + global_coeff)
  z6 = jnp.zeros((B, 32 - T), jnp.float32)
  z70 = jnp.zeros((B, 128 - 32 - T), jnp.float32)
  bc = jnp.concatenate([bsum, z6, csum, z70], axis=1)
  uemb = user_tag_embeddings.reshape(-1, T * D)
  iemb = item_tag_embeddings.reshape(-1, T * D)
  return _run(user, item, uemb, iemb, bc)
